# Initial kernel scaffold; baseline (speedup 1.0000x reference)
import jax
import jax.numpy as jnp
from jax.experimental import pallas as pl


def kernel(fnode, mess_graph, node_wid, agg_graph, embedding, Wz_w, Wz_b, Wr_w, Ur_w, Ur_b, Wh_w, Wh_b, W_w, W_b):
    raise NotImplementedError("write your pallas kernel here")


# speedup vs baseline: 1.7046x; 1.7046x over previous
import functools

import jax
import jax.numpy as jnp
from jax import lax
from jax.experimental import pallas as pl
from jax.experimental.pallas import tpu as pltpu
from jax.experimental.pallas import tpu_sc as plsc

H = 128
E = 20000
N = 10000
EP = 20480
NPAD = 10240
ZROW = 20000
VP = 808
f32 = jnp.float32

_MESH = plsc.VectorSubcoreMesh(core_axis_name="c", subcore_axis_name="s")


def _wid():
    return lax.axis_index("s") * 2 + lax.axis_index("c")


def _prep_body(emb_ref, wz_ref, wzb_ref, wr_ref, ur_ref, urb_ref, wh_ref,
               whb_ref, ww_ref, vhg_ref, vzh_ref, va_ref, vw_ref):
    e = emb_ref[:]
    ez = jnp.dot(e, wz_ref[0:H, :], preferred_element_type=f32) + wzb_ref[0, :]
    eh = jnp.dot(e, wh_ref[0:H, :], preferred_element_type=f32) + whb_ref[0, :]
    hv = jax.nn.sigmoid(ez) * jnp.tanh(eh)
    egv = jnp.exp(-(jnp.dot(hv, ur_ref[:], preferred_element_type=f32)
                    + urb_ref[0, :]))
    expb = jnp.exp(-urb_ref[0, :])
    vhg_ref[0:800, 0:H] = hv
    vhg_ref[0:800, H:] = egv
    vhg_ref[800:VP, 0:H] = jnp.zeros((8, H), f32)
    vhg_ref[800:VP, H:] = jnp.broadcast_to(expb[None, :], (8, H))
    vzh_ref[0:800, 0:H] = ez
    vzh_ref[0:800, H:] = eh
    vzh_ref[800:VP, :] = jnp.zeros((8, 2 * H), f32)
    va_ref[0:800, :] = jnp.exp(-jnp.dot(e, wr_ref[:], preferred_element_type=f32))
    va_ref[800:VP, :] = jnp.ones((8, H), f32)
    vw_ref[0:800, :] = jnp.dot(e, ww_ref[0:H, :], preferred_element_type=f32)
    vw_ref[800:VP, :] = jnp.zeros((8, H), f32)


def _tc_prep(emb, Wz_w, wzb2, Wr_w, Ur_w, urb2, Wh_w, whb2, W_w):
    return pl.pallas_call(
        _prep_body,
        out_shape=[jax.ShapeDtypeStruct((VP, 2 * H), f32),
                   jax.ShapeDtypeStruct((VP, 2 * H), f32),
                   jax.ShapeDtypeStruct((VP, H), f32),
                   jax.ShapeDtypeStruct((VP, H), f32)],
    )(emb, Wz_w, wzb2, Wr_w, Ur_w, urb2, Wh_w, whb2, W_w)


@functools.partial(
    pl.kernel,
    out_type=[jax.ShapeDtypeStruct((EP, 2 * H), f32),
              jax.ShapeDtypeStruct((EP, 2 * H), f32),
              jax.ShapeDtypeStruct((EP, H), f32),
              jax.ShapeDtypeStruct((NPAD, H), f32)],
    mesh=_MESH,
    scratch_types=[pltpu.VMEM((128,), jnp.int32),
                   pltpu.VMEM((128, 2 * H), f32),
                   pltpu.VMEM((128, H), f32),
                   pltpu.SemaphoreType.DMA],
)
def _sc_expand(vhg, vzh, va, vw, fn_ext, nw_ext, hg1, xzh, atab, xw,
               idxb, b256, b128, sem):
    wid = _wid()
    ebase = wid * 640

    def blk(c, carry):
        base = ebase + c * 128
        pltpu.sync_copy(fn_ext.at[pl.ds(base, 128)], idxb)
        pltpu.async_copy(vhg.at[idxb], b256, sem).wait()
        pltpu.sync_copy(b256, hg1.at[pl.ds(base, 128)])
        pltpu.async_copy(vzh.at[idxb], b256, sem).wait()
        pltpu.sync_copy(b256, xzh.at[pl.ds(base, 128)])
        pltpu.async_copy(va.at[idxb], b128, sem).wait()
        pltpu.sync_copy(b128, atab.at[pl.ds(base, 128)])
        return carry

    lax.fori_loop(0, 5, blk, 0)
    nbase = wid * 320

    def blk2(c, carry):
        base = nbase + c * 64
        pltpu.sync_copy(nw_ext.at[pl.ds(base, 64)], idxb.at[pl.ds(0, 64)])
        pltpu.async_copy(vw.at[idxb.at[pl.ds(0, 64)]], b128.at[pl.ds(0, 64)],
                         sem).wait()
        pltpu.sync_copy(b128.at[pl.ds(0, 64)], xw.at[pl.ds(base, 64)])
        return carry

    lax.fori_loop(0, 5, blk2, 0)


@functools.partial(
    pl.kernel,
    out_type=jax.ShapeDtypeStruct((EP, 2 * H), f32),
    mesh=_MESH,
    scratch_types=[pltpu.VMEM((2048,), jnp.int32),
                   pltpu.VMEM((128, H), f32),
                   pltpu.VMEM((128, 2 * H), f32),
                   pltpu.VMEM((128, 2 * H), f32),
                   pltpu.SemaphoreType.DMA],
)
def _sc_msg(hgt, atab, mgf, sums, idxb, ab, nbr, outb, sem):
    wid = _wid()

    def blk(b, carry):
        ebase = wid * 640 + b * 128
        pltpu.sync_copy(mgf.at[pl.ds(ebase * 16, 2048)], idxb)
        pltpu.sync_copy(atab.at[pl.ds(ebase, 128)], ab)

        def chunk(c, carry2):
            pltpu.async_copy(hgt.at[idxb.at[pl.ds(c * 128, 128)]], nbr,
                             sem).wait()

            def edge(e, carry3):
                erow = c * 8 + e
                a = [ab[erow, pl.ds(f * 16, 16)] for f in range(8)]

                def kstep(k, acc):
                    row = e * 16 + k
                    out = list(acc)
                    for f in range(8):
                        hv = nbr[row, pl.ds(f * 16, 16)]
                        gv = nbr[row, pl.ds(H + f * 16, 16)]
                        out[f] = acc[f] + hv
                        out[8 + f] = acc[8 + f] + hv / (a[f] * gv + 1.0)
                    return tuple(out)

                acc0 = tuple(jnp.zeros((16,), f32) for _ in range(16))
                acc = lax.fori_loop(0, 16, kstep, acc0)
                for f in range(8):
                    outb[erow, pl.ds(f * 16, 16)] = acc[f]
                    outb[erow, pl.ds(H + f * 16, 16)] = acc[8 + f]
                return carry3

            lax.fori_loop(0, 8, edge, 0)
            return carry2

        lax.fori_loop(0, 16, chunk, 0)
        pltpu.sync_copy(outb, sums.at[pl.ds(ebase, 128)])
        return carry

    lax.fori_loop(0, 5, blk, 0)


@functools.partial(
    pl.kernel,
    out_type=jax.ShapeDtypeStruct((NPAD, H), f32),
    mesh=_MESH,
    scratch_types=[pltpu.VMEM((1024,), jnp.int32),
                   pltpu.VMEM((128, H), f32),
                   pltpu.VMEM((64, H), f32),
                   pltpu.SemaphoreType.DMA],
)
def _sc_agg(h4, agf, sumn, idxb, nbr, outb, sem):
    wid = _wid()

    def blk(b, carry):
        nbase = wid * 320 + b * 64
        pltpu.sync_copy(agf.at[pl.ds(nbase * 16, 1024)], idxb)

        def chunk(c, carry2):
            pltpu.async_copy(h4.at[idxb.at[pl.ds(c * 128, 128)]], nbr,
                             sem).wait()

            def node(e, carry3):
                nrow = c * 8 + e

                def kstep(k, acc):
                    row = e * 16 + k
                    return tuple(acc[f] + nbr[row, pl.ds(f * 16, 16)]
                                 for f in range(8))

                acc0 = tuple(jnp.zeros((16,), f32) for _ in range(8))
                acc = lax.fori_loop(0, 16, kstep, acc0)
                for f in range(8):
                    outb[nrow, pl.ds(f * 16, 16)] = acc[f]
                return carry3

            lax.fori_loop(0, 8, node, 0)
            return carry2

        lax.fori_loop(0, 8, chunk, 0)
        pltpu.sync_copy(outb, sumn.at[pl.ds(nbase, 64)])
        return carry

    lax.fori_loop(0, 5, blk, 0)


def _upd_body(emit_eg, sums_ref, xzh_ref, wzb_ref, whb_ref, ur_ref, urb_ref,
              out_ref):
    sh = sums_ref[:, 0:H]
    sq = sums_ref[:, H:]
    z = jax.nn.sigmoid(xzh_ref[:, 0:H]
                       + jnp.dot(sh, wzb_ref[:], preferred_element_type=f32))
    pre = jnp.tanh(xzh_ref[:, H:]
                   + jnp.dot(sq, whb_ref[:], preferred_element_type=f32))
    hn = (1.0 - z) * sh + z * pre
    rowid = (pl.program_id(0) * 512
             + lax.broadcasted_iota(jnp.int32, (512, 1), 0))
    hn = jnp.where(rowid == ZROW, 0.0, hn)
    if emit_eg:
        eg = jnp.exp(-(jnp.dot(hn, ur_ref[:], preferred_element_type=f32)
                       + urb_ref[0, :]))
        out_ref[:, 0:H] = hn
        out_ref[:, H:] = eg
    else:
        out_ref[:] = hn


def _tc_upd(sums, xzh, Wzb, Whb, Ur_w, urb2, emit_eg):
    width = 2 * H if emit_eg else H
    wspec = pl.BlockSpec((H, H), lambda i: (0, 0))
    return pl.pallas_call(
        functools.partial(_upd_body, emit_eg),
        grid=(EP // 512,),
        in_specs=[pl.BlockSpec((512, 2 * H), lambda i: (i, 0)),
                  pl.BlockSpec((512, 2 * H), lambda i: (i, 0)),
                  wspec, wspec, wspec,
                  pl.BlockSpec((1, H), lambda i: (0, 0))],
        out_specs=pl.BlockSpec((512, width), lambda i: (i, 0)),
        out_shape=jax.ShapeDtypeStruct((EP, width), f32),
    )(sums, xzh, Wzb, Whb, Ur_w, urb2)


def _fin_body(xw_ref, sumn_ref, wb_ref, b_ref, out_ref):
    acc = (xw_ref[:]
           + jnp.dot(sumn_ref[:], wb_ref[:], preferred_element_type=f32)
           + b_ref[0, :])
    out_ref[:] = jnp.maximum(acc, 0.0)


def _tc_fin(xw, sumn, Wb, wb2):
    return pl.pallas_call(
        _fin_body,
        grid=(NPAD // 512,),
        in_specs=[pl.BlockSpec((512, H), lambda i: (i, 0)),
                  pl.BlockSpec((512, H), lambda i: (i, 0)),
                  pl.BlockSpec((H, H), lambda i: (0, 0)),
                  pl.BlockSpec((1, H), lambda i: (0, 0))],
        out_specs=pl.BlockSpec((512, H), lambda i: (i, 0)),
        out_shape=jax.ShapeDtypeStruct((NPAD, H), f32),
    )(xw, sumn, Wb, wb2)


def kernel(fnode, mess_graph, node_wid, agg_graph, embedding, Wz_w, Wz_b,
           Wr_w, Ur_w, Ur_b, Wh_w, Wh_b, W_w, W_b):
    fnode = fnode.astype(jnp.int32)
    node_wid = node_wid.astype(jnp.int32)
    mess_graph = mess_graph.astype(jnp.int32)
    agg_graph = agg_graph.astype(jnp.int32)

    mg_r = jnp.where(mess_graph == 0, ZROW, mess_graph - 1)
    mgf = jnp.concatenate(
        [mg_r, jnp.full((EP - E, 16), ZROW, jnp.int32)], 0).reshape(-1)
    ag_r = jnp.where(agg_graph == 0, ZROW, agg_graph - 1)
    agf = jnp.concatenate(
        [ag_r, jnp.full((NPAD - N, 16), ZROW, jnp.int32)], 0).reshape(-1)
    fn_ext = jnp.concatenate([fnode, jnp.full((EP - E,), 800, jnp.int32)])
    nw_ext = jnp.concatenate([node_wid, jnp.zeros((NPAD - N,), jnp.int32)])

    wzb2 = Wz_b.reshape(1, H)
    whb2 = Wh_b.reshape(1, H)
    urb2 = Ur_b.reshape(1, H)
    wb2 = W_b.reshape(1, H)
    Wzb = Wz_w[H:]
    Whb = Wh_w[H:]
    Wb = W_w[H:]

    vhg, vzh, va, vw = _tc_prep(embedding, Wz_w, wzb2, Wr_w, Ur_w, urb2,
                                Wh_w, whb2, W_w)
    hg, xzh, atab, xw = _sc_expand(vhg, vzh, va, vw, fn_ext, nw_ext)

    for _ in (2, 3):
        sums = _sc_msg(hg, atab, mgf)
        hg = _tc_upd(sums, xzh, Wzb, Whb, Ur_w, urb2, True)
    sums = _sc_msg(hg, atab, mgf)
    h4 = _tc_upd(sums, xzh, Wzb, Whb, Ur_w, urb2, False)

    sumn = _sc_agg(h4, agf)
    out = _tc_fin(xw, sumn, Wb, wb2)
    return out[:N]
